# Initial kernel scaffold; baseline (speedup 1.0000x reference)
import jax
import jax.numpy as jnp
from jax.experimental import pallas as pl


def kernel(x, pos, edge_index):
    raise NotImplementedError("write your pallas kernel here")


# speedup vs baseline: 87.1677x; 87.1677x over previous
import functools

import jax
import jax.numpy as jnp
from jax import lax
from jax.experimental import pallas as pl
from jax.experimental.pallas import tpu as pltpu
from jax.experimental.pallas import tpu_sc as plsc

N_NODES = 10000
N_PAD = 10240
E_EDGES = 320000
NUM_CORES = 2
NUM_SUBCORES = 16
NUM_WORKERS = NUM_CORES * NUM_SUBCORES
E_PER_WORKER = E_EDGES // NUM_WORKERS
LANES = 16
WEIGHT_CONST = 0.01


def _sc_body(phi_hbm, px_hbm, py_hbm, pz_hbm, src_hbm, dst_hbm, out_hbm,
             phi_v, px_v, py_v, pz_v, src_v, dst_v, lap_v, deg_v):
    wid = lax.axis_index("s") * NUM_CORES + lax.axis_index("c")
    base = wid * E_PER_WORKER

    pltpu.sync_copy(phi_hbm, phi_v)
    pltpu.sync_copy(px_hbm, px_v)
    pltpu.sync_copy(py_hbm, py_v)
    pltpu.sync_copy(pz_hbm, pz_v)
    pltpu.sync_copy(src_hbm.at[pl.ds(base, E_PER_WORKER)], src_v)
    pltpu.sync_copy(dst_hbm.at[pl.ds(base, E_PER_WORKER)], dst_v)

    zeros = jnp.zeros((LANES,), jnp.float32)

    def zero_body(j, carry):
        lap_v[pl.ds(j * LANES, LANES)] = zeros
        deg_v[pl.ds(j * LANES, LANES)] = zeros
        return carry

    lax.fori_loop(0, N_PAD // LANES, zero_body, 0)

    ones = jnp.ones((LANES,), jnp.float32)

    def edge_body(i, carry):
        o = i * LANES
        s = src_v[pl.ds(o, LANES)]
        d = dst_v[pl.ds(o, LANES)]
        dphi = plsc.load_gather(phi_v, [s]) - plsc.load_gather(phi_v, [d])
        dx = plsc.load_gather(px_v, [s]) - plsc.load_gather(px_v, [d])
        dy = plsc.load_gather(py_v, [s]) - plsc.load_gather(py_v, [d])
        dz = plsc.load_gather(pz_v, [s]) - plsc.load_gather(pz_v, [d])
        dist2 = dx * dx + dy * dy + dz * dz + 1e-8
        plsc.addupdate_scatter(lap_v, [d], dphi / dist2)
        plsc.addupdate_scatter(deg_v, [d], ones)
        return carry

    lax.fori_loop(0, E_PER_WORKER // LANES, edge_body, 0)

    pltpu.sync_copy(lap_v, out_hbm.at[wid])
    pltpu.sync_copy(deg_v, out_hbm.at[NUM_WORKERS + wid])


_sc_partials = functools.partial(
    pl.kernel,
    out_type=jax.ShapeDtypeStruct((2 * NUM_WORKERS, N_PAD), jnp.float32),
    mesh=plsc.VectorSubcoreMesh(core_axis_name="c", subcore_axis_name="s",
                                num_cores=NUM_CORES,
                                num_subcores=NUM_SUBCORES),
    compiler_params=pltpu.CompilerParams(needs_layout_passes=False),
    scratch_types=[
        pltpu.VMEM((N_PAD,), jnp.float32),
        pltpu.VMEM((N_PAD,), jnp.float32),
        pltpu.VMEM((N_PAD,), jnp.float32),
        pltpu.VMEM((N_PAD,), jnp.float32),
        pltpu.VMEM((E_PER_WORKER,), jnp.int32),
        pltpu.VMEM((E_PER_WORKER,), jnp.int32),
        pltpu.VMEM((N_PAD,), jnp.float32),
        pltpu.VMEM((N_PAD,), jnp.float32),
    ],
)(_sc_body)


def _finalize_body(part_ref, out_ref):
    part = part_ref[...]
    lap = jnp.sum(part[:NUM_WORKERS, :], axis=0)
    deg = jnp.sum(part[NUM_WORKERS:, :], axis=0)
    curv = lap / (deg + 1e-8)
    out_ref[0, 0] = WEIGHT_CONST * jnp.sum(curv * curv) / float(N_NODES)


def kernel(x, pos, edge_index):
    phi = jnp.pad(x[:, 8], (0, N_PAD - N_NODES))
    posp = jnp.pad(pos, ((0, N_PAD - N_NODES), (0, 0)))
    px, py, pz = posp[:, 0], posp[:, 1], posp[:, 2]
    src = edge_index[0]
    dst = edge_index[1]

    part = _sc_partials(phi, px, py, pz, src, dst)

    loss = pl.pallas_call(
        _finalize_body,
        out_shape=jax.ShapeDtypeStruct((1, 1), jnp.float32),
        in_specs=[pl.BlockSpec((2 * NUM_WORKERS, N_PAD), lambda: (0, 0))],
        out_specs=pl.BlockSpec(memory_space=pltpu.SMEM),
    )(part)
    return jnp.reshape(loss, ())


# speedup vs baseline: 118.4477x; 1.3588x over previous
import functools

import jax
import jax.numpy as jnp
from jax import lax
from jax.experimental import pallas as pl
from jax.experimental.pallas import tpu as pltpu
from jax.experimental.pallas import tpu_sc as plsc

N_NODES = 10000
E_EDGES = 320000
NUM_CORES = 2
NUM_SUBCORES = 16
NUM_WORKERS = NUM_CORES * NUM_SUBCORES
E_PER_WORKER = E_EDGES // NUM_WORKERS
LANES = 16
WEIGHT_CONST = 0.01


def _sc_body(phi_hbm, pos_hbm, ei_hbm, out_hbm,
             phi_v, pos_v, src_v, dst_v, lap_v, deg_v, sem):
    wid = lax.axis_index("s") * NUM_CORES + lax.axis_index("c")
    base = wid * E_PER_WORKER

    c0 = pltpu.async_copy(phi_hbm, phi_v, sem)
    c1 = pltpu.async_copy(pos_hbm, pos_v, sem)
    c2 = pltpu.async_copy(ei_hbm.at[pl.ds(base, E_PER_WORKER)], src_v, sem)
    c3 = pltpu.async_copy(ei_hbm.at[pl.ds(E_EDGES + base, E_PER_WORKER)],
                          dst_v, sem)

    zeros = jnp.zeros((LANES,), jnp.float32)

    @plsc.parallel_loop(0, N_NODES // LANES, unroll=8)
    def _zero(j):
        lap_v[pl.ds(j * LANES, LANES)] = zeros
        deg_v[pl.ds(j * LANES, LANES)] = zeros

    c0.wait()
    c1.wait()
    c2.wait()
    c3.wait()

    ones = jnp.ones((LANES,), jnp.float32)

    @plsc.parallel_loop(0, E_PER_WORKER // LANES, unroll=8)
    def _edges(i):
        o = i * LANES
        s = src_v[pl.ds(o, LANES)]
        d = dst_v[pl.ds(o, LANES)]
        dphi = plsc.load_gather(phi_v, [s]) - plsc.load_gather(phi_v, [d])
        s3 = s * 3
        d3 = d * 3
        dx = plsc.load_gather(pos_v, [s3]) - plsc.load_gather(pos_v, [d3])
        dy = plsc.load_gather(pos_v, [s3 + 1]) - plsc.load_gather(pos_v, [d3 + 1])
        dz = plsc.load_gather(pos_v, [s3 + 2]) - plsc.load_gather(pos_v, [d3 + 2])
        dist2 = dx * dx + dy * dy + dz * dz + 1e-8
        plsc.addupdate_scatter(lap_v, [d], dphi / dist2)
        plsc.addupdate_scatter(deg_v, [d], ones)

    o0 = pltpu.async_copy(lap_v, out_hbm.at[wid], sem)
    o1 = pltpu.async_copy(deg_v, out_hbm.at[NUM_WORKERS + wid], sem)
    o0.wait()
    o1.wait()


_sc_partials = functools.partial(
    pl.kernel,
    out_type=jax.ShapeDtypeStruct((2 * NUM_WORKERS, N_NODES), jnp.float32),
    mesh=plsc.VectorSubcoreMesh(core_axis_name="c", subcore_axis_name="s",
                                num_cores=NUM_CORES,
                                num_subcores=NUM_SUBCORES),
    compiler_params=pltpu.CompilerParams(needs_layout_passes=False),
    scratch_types=[
        pltpu.VMEM((N_NODES,), jnp.float32),
        pltpu.VMEM((3 * N_NODES,), jnp.float32),
        pltpu.VMEM((E_PER_WORKER,), jnp.int32),
        pltpu.VMEM((E_PER_WORKER,), jnp.int32),
        pltpu.VMEM((N_NODES,), jnp.float32),
        pltpu.VMEM((N_NODES,), jnp.float32),
        pltpu.SemaphoreType.DMA,
    ],
)(_sc_body)


def _finalize_body(part_ref, out_ref):
    part = part_ref[...]
    lap = jnp.sum(part[:NUM_WORKERS, :], axis=0)
    deg = jnp.sum(part[NUM_WORKERS:, :], axis=0)
    curv = lap / (deg + 1e-8)
    out_ref[0, 0] = WEIGHT_CONST * jnp.sum(curv * curv) / float(N_NODES)


def kernel(x, pos, edge_index):
    phi = x[:, 8]
    pos_flat = jnp.reshape(pos, (-1,))
    ei_flat = jnp.reshape(edge_index, (-1,))

    part = _sc_partials(phi, pos_flat, ei_flat)

    loss = pl.pallas_call(
        _finalize_body,
        out_shape=jax.ShapeDtypeStruct((1, 1), jnp.float32),
        in_specs=[pl.BlockSpec((2 * NUM_WORKERS, N_NODES), lambda: (0, 0))],
        out_specs=pl.BlockSpec(memory_space=pltpu.SMEM),
    )(part)
    return jnp.reshape(loss, ())


# speedup vs baseline: 127.4361x; 1.0759x over previous
import functools

import jax
import jax.numpy as jnp
from jax import lax
from jax.experimental import pallas as pl
from jax.experimental.pallas import tpu as pltpu
from jax.experimental.pallas import tpu_sc as plsc

N_NODES = 10000
E_EDGES = 320000
NUM_CORES = 2
NUM_SUBCORES = 16
NUM_WORKERS = NUM_CORES * NUM_SUBCORES
E_PER_WORKER = E_EDGES // NUM_WORKERS
LANES = 16
WEIGHT_CONST = 0.01


def _sc_body(phi_hbm, pos_hbm, ei_hbm, out_hbm,
             phi_v, pos_v, src_v, dst_v, lap_v, deg_v,
             phi_sh, pos_sh, sem):
    cid = lax.axis_index("c")
    sid = lax.axis_index("s")
    wid = sid * NUM_CORES + cid
    base = wid * E_PER_WORKER

    with jax.named_scope("stage_shared"):
        @pl.when(sid == 0)
        def _():
            pltpu.sync_copy(phi_hbm, phi_sh)

        @pl.when(sid == 1)
        def _():
            pltpu.sync_copy(pos_hbm, pos_sh)

    c2 = pltpu.async_copy(ei_hbm.at[pl.ds(base, E_PER_WORKER)], src_v, sem)
    c3 = pltpu.async_copy(ei_hbm.at[pl.ds(E_EDGES + base, E_PER_WORKER)],
                          dst_v, sem)

    zeros = jnp.zeros((LANES,), jnp.float32)

    with jax.named_scope("zero"):
        @plsc.parallel_loop(0, N_NODES // LANES, unroll=8)
        def _zero(j):
            lap_v[pl.ds(j * LANES, LANES)] = zeros
            deg_v[pl.ds(j * LANES, LANES)] = zeros

    plsc.subcore_barrier()

    with jax.named_scope("stage_local"):
        pltpu.sync_copy(phi_sh, phi_v)
        pltpu.sync_copy(pos_sh, pos_v)
        c2.wait()
        c3.wait()

    ones = jnp.ones((LANES,), jnp.float32)

    with jax.named_scope("edges"):
        @plsc.parallel_loop(0, E_PER_WORKER // LANES, unroll=8)
        def _edges(i):
            o = i * LANES
            s = src_v[pl.ds(o, LANES)]
            d = dst_v[pl.ds(o, LANES)]
            dphi = plsc.load_gather(phi_v, [s]) - plsc.load_gather(phi_v, [d])
            s3 = s * 3
            d3 = d * 3
            dx = plsc.load_gather(pos_v, [s3]) - plsc.load_gather(pos_v, [d3])
            dy = plsc.load_gather(pos_v, [s3 + 1]) - plsc.load_gather(pos_v, [d3 + 1])
            dz = plsc.load_gather(pos_v, [s3 + 2]) - plsc.load_gather(pos_v, [d3 + 2])
            dist2 = dx * dx + dy * dy + dz * dz + 1e-8
            plsc.addupdate_scatter(lap_v, [d], dphi / dist2)
            plsc.addupdate_scatter(deg_v, [d], ones)

    with jax.named_scope("out"):
        o0 = pltpu.async_copy(lap_v, out_hbm.at[wid], sem)
        o1 = pltpu.async_copy(deg_v, out_hbm.at[NUM_WORKERS + wid], sem)
        o0.wait()
        o1.wait()


_sc_partials = functools.partial(
    pl.kernel,
    out_type=jax.ShapeDtypeStruct((2 * NUM_WORKERS, N_NODES), jnp.float32),
    mesh=plsc.VectorSubcoreMesh(core_axis_name="c", subcore_axis_name="s",
                                num_cores=NUM_CORES,
                                num_subcores=NUM_SUBCORES),
    compiler_params=pltpu.CompilerParams(needs_layout_passes=False),
    scratch_types=[
        pltpu.VMEM((N_NODES,), jnp.float32),
        pltpu.VMEM((3 * N_NODES,), jnp.float32),
        pltpu.VMEM((E_PER_WORKER,), jnp.int32),
        pltpu.VMEM((E_PER_WORKER,), jnp.int32),
        pltpu.VMEM((N_NODES,), jnp.float32),
        pltpu.VMEM((N_NODES,), jnp.float32),
        pltpu.VMEM_SHARED((N_NODES,), jnp.float32),
        pltpu.VMEM_SHARED((3 * N_NODES,), jnp.float32),
        pltpu.SemaphoreType.DMA,
    ],
)(_sc_body)


def _finalize_body(part_ref, out_ref):
    part = part_ref[...]
    lap = jnp.sum(part[:NUM_WORKERS, :], axis=0)
    deg = jnp.sum(part[NUM_WORKERS:, :], axis=0)
    curv = lap / (deg + 1e-8)
    out_ref[0, 0] = WEIGHT_CONST * jnp.sum(curv * curv) / float(N_NODES)


def kernel(x, pos, edge_index):
    phi = x[:, 8]
    pos_flat = jnp.reshape(pos, (-1,))
    ei_flat = jnp.reshape(edge_index, (-1,))

    part = _sc_partials(phi, pos_flat, ei_flat)

    loss = pl.pallas_call(
        _finalize_body,
        out_shape=jax.ShapeDtypeStruct((1, 1), jnp.float32),
        in_specs=[pl.BlockSpec((2 * NUM_WORKERS, N_NODES), lambda: (0, 0))],
        out_specs=pl.BlockSpec(memory_space=pltpu.SMEM),
    )(part)
    return jnp.reshape(loss, ())


# speedup vs baseline: 138.5997x; 1.0876x over previous
import functools

import jax
import jax.numpy as jnp
from jax import lax
from jax.experimental import pallas as pl
from jax.experimental.pallas import tpu as pltpu
from jax.experimental.pallas import tpu_sc as plsc

N_NODES = 10000
E_EDGES = 320000
E_COLS = E_EDGES // 128
NUM_CORES = 2
NUM_SUBCORES = 16
NUM_WORKERS = NUM_CORES * NUM_SUBCORES
E_PER_WORKER = E_EDGES // NUM_WORKERS
WINDOW_COLS = E_COLS // NUM_WORKERS + 1
WINDOW = WINDOW_COLS * 128
LANES = 16
WEIGHT_CONST = 0.01


def _sc_body(phi_hbm, pos_hbm, ei_hbm, out_hbm,
             phi_v, pos_v, ei_v, lap_v, deg_v,
             phi_sh, pos_sh, sem):
    cid = lax.axis_index("c")
    sid = lax.axis_index("s")
    wid = sid * NUM_CORES + cid
    base = wid * E_PER_WORKER
    col0 = (wid * E_COLS) // NUM_WORKERS
    off = base - col0 * 128

    with jax.named_scope("stage_shared"):
        @pl.when(sid == 0)
        def _():
            pltpu.sync_copy(phi_hbm, phi_sh)

        @pl.when(sid == 1)
        def _():
            pltpu.sync_copy(pos_hbm, pos_sh)

    c2 = pltpu.async_copy(
        ei_hbm.at[pl.ds(0, 2), pl.ds(col0 * 128, WINDOW)], ei_v, sem)

    zeros = jnp.zeros((LANES,), jnp.float32)

    with jax.named_scope("zero"):
        @plsc.parallel_loop(0, N_NODES // LANES, unroll=8)
        def _zero(j):
            lap_v[pl.ds(j * LANES, LANES)] = zeros
            deg_v[pl.ds(j * LANES, LANES)] = zeros

    plsc.subcore_barrier()

    with jax.named_scope("stage_local"):
        pltpu.sync_copy(phi_sh, phi_v)
        pltpu.sync_copy(pos_sh, pos_v)
        c2.wait()

    ones = jnp.ones((LANES,), jnp.float32)

    with jax.named_scope("edges"):
        @plsc.parallel_loop(0, E_PER_WORKER // LANES, unroll=8)
        def _edges(i):
            o = off + i * LANES
            s = ei_v[0, pl.ds(o, LANES)]
            d = ei_v[1, pl.ds(o, LANES)]
            dphi = plsc.load_gather(phi_v, [s]) - plsc.load_gather(phi_v, [d])
            s3 = s * 3
            d3 = d * 3
            dx = plsc.load_gather(pos_v, [s3]) - plsc.load_gather(pos_v, [d3])
            dy = plsc.load_gather(pos_v, [s3 + 1]) - plsc.load_gather(pos_v, [d3 + 1])
            dz = plsc.load_gather(pos_v, [s3 + 2]) - plsc.load_gather(pos_v, [d3 + 2])
            dist2 = dx * dx + dy * dy + dz * dz + 1e-8
            plsc.addupdate_scatter(lap_v, [d], dphi / dist2)
            plsc.addupdate_scatter(deg_v, [d], ones)

    with jax.named_scope("out"):
        o0 = pltpu.async_copy(lap_v, out_hbm.at[wid], sem)
        o1 = pltpu.async_copy(deg_v, out_hbm.at[NUM_WORKERS + wid], sem)
        o0.wait()
        o1.wait()


_sc_partials = functools.partial(
    pl.kernel,
    out_type=jax.ShapeDtypeStruct((2 * NUM_WORKERS, N_NODES), jnp.float32),
    mesh=plsc.VectorSubcoreMesh(core_axis_name="c", subcore_axis_name="s",
                                num_cores=NUM_CORES,
                                num_subcores=NUM_SUBCORES),
    compiler_params=pltpu.CompilerParams(needs_layout_passes=False),
    scratch_types=[
        pltpu.VMEM((N_NODES,), jnp.float32),
        pltpu.VMEM((3 * N_NODES,), jnp.float32),
        pltpu.VMEM((2, WINDOW), jnp.int32),
        pltpu.VMEM((N_NODES,), jnp.float32),
        pltpu.VMEM((N_NODES,), jnp.float32),
        pltpu.VMEM_SHARED((N_NODES,), jnp.float32),
        pltpu.VMEM_SHARED((3 * N_NODES,), jnp.float32),
        pltpu.SemaphoreType.DMA,
    ],
)(_sc_body)


def _finalize_body(part_ref, out_ref):
    part = part_ref[...]
    lap = jnp.sum(part[:NUM_WORKERS, :], axis=0)
    deg = jnp.sum(part[NUM_WORKERS:, :], axis=0)
    curv = lap / (deg + 1e-8)
    out_ref[0, 0] = WEIGHT_CONST * jnp.sum(curv * curv) / float(N_NODES)


def kernel(x, pos, edge_index):
    phi = x[:, 8]
    pos_flat = jnp.reshape(pos, (-1,))

    part = _sc_partials(phi, pos_flat, edge_index)

    loss = pl.pallas_call(
        _finalize_body,
        out_shape=jax.ShapeDtypeStruct((1, 1), jnp.float32),
        in_specs=[pl.BlockSpec((2 * NUM_WORKERS, N_NODES), lambda: (0, 0))],
        out_specs=pl.BlockSpec(memory_space=pltpu.SMEM),
    )(part)
    return jnp.reshape(loss, ())
